# baseline (speedup 1.0000x reference)
import functools

import jax
import jax.numpy as jnp
from jax import lax
from jax.experimental import pallas as pl
from jax.experimental.pallas import tpu as pltpu
from jax.experimental.pallas import tpu_sc as plsc

K = 512
C = 32
N = 4096
BLK = 2048
GRID = N // BLK

NC = 2
NS = 16
NW = NC * NS
BPW = N // NW

_BF = jnp.bfloat16
_F = jnp.float32


def _bdot(a, b):
    return lax.dot_general(a, b, (((1,), (0,)), ((), ())),
                           preferred_element_type=_F)


def _argmin_body(x_ref, e_ref, idx_ref,
                 eth_s, etl_s, eh_s, em_s, el_s, e2_s):
    @pl.when(pl.program_id(0) == 0)
    def _prep():
        e = e_ref[...]
        et = e.T
        eth = et.astype(_BF)
        etl = (et - eth.astype(_F)).astype(_BF)
        eth_s[...] = eth
        etl_s[...] = etl
        eh = e.astype(_BF)
        r = e - eh.astype(_F)
        em = r.astype(_BF)
        el = (r - em.astype(_F)).astype(_BF)
        eh_s[...] = eh
        em_s[...] = em
        el_s[...] = el
        e2_s[...] = jnp.sum(et * et, axis=0)[None, :]

    x = x_ref[...]
    xh = x.astype(_BF)
    xl = (x - xh.astype(_F)).astype(_BF)
    eth = eth_s[...]
    etl = etl_s[...]
    s = _bdot(xh, eth) + _bdot(xh, etl) + _bdot(xl, eth)
    p = e2_s[...] - 2.0 * s
    kio = lax.broadcasted_iota(jnp.int32, (BLK, K), 1)
    v1 = jnp.min(p, axis=1)
    i1 = jnp.min(jnp.where(p == v1[:, None], kio, K), axis=1)
    m1 = kio == i1[:, None]
    p2m = jnp.where(m1, jnp.inf, p)
    v2 = jnp.min(p2m, axis=1)
    i2 = jnp.min(jnp.where(p2m == v2[:, None], kio, K), axis=1)
    m2 = kio == i2[:, None]
    m1b = m1.astype(_BF)
    m2b = m2.astype(_BF)
    eh = eh_s[...]
    em = em_s[...]
    el = el_s[...]
    e1 = (_bdot(m1b, eh) + _bdot(m1b, em)) + _bdot(m1b, el)
    e2v = (_bdot(m2b, eh) + _bdot(m2b, em)) + _bdot(m2b, el)
    d1 = jnp.sum((x - e1) ** 2, axis=1)
    d2 = jnp.sum((x - e2v) ** 2, axis=1)
    pick = jnp.where(d2 < d1, i2, jnp.where(d1 < d2, i1, jnp.minimum(i1, i2)))
    idx_ref[0, 0, :] = pick


def _nearest_indices(flat, embedding, interpret=False):
    return pl.pallas_call(
        _argmin_body,
        grid=(GRID,),
        in_specs=[
            pl.BlockSpec((BLK, C), lambda i: (i, 0)),
            pl.BlockSpec((K, C), lambda i: (0, 0)),
        ],
        out_specs=pl.BlockSpec((1, 1, BLK), lambda i: (i, 0, 0)),
        out_shape=jax.ShapeDtypeStruct((GRID, 1, BLK), jnp.int32),
        scratch_shapes=[
            pltpu.VMEM((C, K), _BF),
            pltpu.VMEM((C, K), _BF),
            pltpu.VMEM((K, C), _BF),
            pltpu.VMEM((K, C), _BF),
            pltpu.VMEM((K, C), _BF),
            pltpu.VMEM((1, K), _F),
        ],
        interpret=interpret,
    )(flat, embedding)


def _sc_gather_body(emb_hbm, idx_hbm, out_hbm, idx_v, rows_v, sem):
    wid = lax.axis_index("s") * NC + lax.axis_index("c")
    base = wid * BPW
    pltpu.sync_copy(idx_hbm.at[pl.ds(base, BPW)], idx_v)
    pltpu.async_copy(emb_hbm.at[idx_v], rows_v, sem).wait()
    pltpu.sync_copy(rows_v, out_hbm.at[pl.ds(base, BPW)])


@functools.cache
def _sc_gather():
    return pl.kernel(
        _sc_gather_body,
        out_type=jax.ShapeDtypeStruct((N, C), jnp.float32),
        mesh=plsc.VectorSubcoreMesh(core_axis_name="c", subcore_axis_name="s",
                                    num_cores=NC, num_subcores=NS),
        scratch_types=[
            pltpu.VMEM((BPW,), jnp.int32),
            pltpu.VMEM((BPW, C), jnp.float32),
            pltpu.SemaphoreType.DMA,
        ],
        compiler_params=pltpu.CompilerParams(use_tc_tiling_on_sc=False),
    )


def kernel(input, embedding):
    B, w, h, c = input.shape
    flat = input.reshape(B * w * h, c)
    idx = _nearest_indices(flat, embedding).reshape(-1)
    q = _sc_gather()(embedding, idx)
    return q.reshape(B, w, h, c)


# speedup vs baseline: 1.0398x; 1.0398x over previous
import functools

import jax
import jax.numpy as jnp
from jax import lax
from jax.experimental import pallas as pl
from jax.experimental.pallas import tpu as pltpu
from jax.experimental.pallas import tpu_sc as plsc

K = 512
C = 32
N = 4096
BLK = 1024
GRID = N // BLK

NC = 2
NS = 16
NW = NC * NS
BPW = N // NW

_BF = jnp.bfloat16
_F = jnp.float32


def _bdot(a, b):
    return lax.dot_general(a, b, (((1,), (0,)), ((), ())),
                           preferred_element_type=_F)


def _argmin_body(x_ref, e_ref, idx_ref,
                 eth_s, etl_s, eh_s, em_s, el_s, e2_s):
    @pl.when(pl.program_id(0) == 0)
    def _prep():
        e = e_ref[...]
        et = e.T
        eth = et.astype(_BF)
        etl = (et - eth.astype(_F)).astype(_BF)
        eth_s[...] = eth
        etl_s[...] = etl
        eh = e.astype(_BF)
        r = e - eh.astype(_F)
        em = r.astype(_BF)
        el = (r - em.astype(_F)).astype(_BF)
        eh_s[...] = eh
        em_s[...] = em
        el_s[...] = el
        e2_s[...] = jnp.sum(et * et, axis=0)[None, :]

    x = x_ref[...]
    xh = x.astype(_BF)
    xl = (x - xh.astype(_F)).astype(_BF)
    eth = eth_s[...]
    etl = etl_s[...]
    s = _bdot(xh, eth) + _bdot(xh, etl) + _bdot(xl, eth)
    p = e2_s[...] - 2.0 * s
    kio = lax.broadcasted_iota(jnp.int32, (BLK, K), 1)
    v1 = jnp.min(p, axis=1)
    i1 = jnp.min(jnp.where(p == v1[:, None], kio, K), axis=1)
    m1 = kio == i1[:, None]
    p2m = jnp.where(m1, jnp.inf, p)
    v2 = jnp.min(p2m, axis=1)
    i2 = jnp.min(jnp.where(p2m == v2[:, None], kio, K), axis=1)
    m2 = kio == i2[:, None]
    mb = jnp.concatenate([m1.astype(_BF), m2.astype(_BF)], axis=0)
    eh = eh_s[...]
    em = em_s[...]
    el = el_s[...]
    e12 = (_bdot(mb, eh) + _bdot(mb, em)) + _bdot(mb, el)
    e1 = e12[:BLK]
    e2v = e12[BLK:]
    d1 = jnp.sum((x - e1) ** 2, axis=1)
    d2 = jnp.sum((x - e2v) ** 2, axis=1)
    pick = jnp.where(d2 < d1, i2, jnp.where(d1 < d2, i1, jnp.minimum(i1, i2)))
    idx_ref[0, 0, :] = pick


def _nearest_indices(flat, embedding, interpret=False):
    return pl.pallas_call(
        _argmin_body,
        grid=(GRID,),
        in_specs=[
            pl.BlockSpec((BLK, C), lambda i: (i, 0)),
            pl.BlockSpec((K, C), lambda i: (0, 0)),
        ],
        out_specs=pl.BlockSpec((1, 1, BLK), lambda i: (i, 0, 0)),
        out_shape=jax.ShapeDtypeStruct((GRID, 1, BLK), jnp.int32),
        scratch_shapes=[
            pltpu.VMEM((C, K), _BF),
            pltpu.VMEM((C, K), _BF),
            pltpu.VMEM((K, C), _BF),
            pltpu.VMEM((K, C), _BF),
            pltpu.VMEM((K, C), _BF),
            pltpu.VMEM((1, K), _F),
        ],
        interpret=interpret,
    )(flat, embedding)


def _sc_gather_body(emb_hbm, idx_hbm, out_hbm, idx_v, rows_v, sem):
    wid = lax.axis_index("s") * NC + lax.axis_index("c")
    base = wid * BPW
    pltpu.sync_copy(idx_hbm.at[pl.ds(base, BPW)], idx_v)
    pltpu.async_copy(emb_hbm.at[idx_v], rows_v, sem).wait()
    pltpu.sync_copy(rows_v, out_hbm.at[pl.ds(base, BPW)])


@functools.cache
def _sc_gather():
    return pl.kernel(
        _sc_gather_body,
        out_type=jax.ShapeDtypeStruct((N, C), jnp.float32),
        mesh=plsc.VectorSubcoreMesh(core_axis_name="c", subcore_axis_name="s",
                                    num_cores=NC, num_subcores=NS),
        scratch_types=[
            pltpu.VMEM((BPW,), jnp.int32),
            pltpu.VMEM((BPW, C), jnp.float32),
            pltpu.SemaphoreType.DMA,
        ],
        compiler_params=pltpu.CompilerParams(use_tc_tiling_on_sc=False),
    )


def kernel(input, embedding):
    B, w, h, c = input.shape
    flat = input.reshape(B * w * h, c)
    idx = _nearest_indices(flat, embedding).reshape(-1)
    q = _sc_gather()(embedding, idx)
    return q.reshape(B, w, h, c)


# speedup vs baseline: 1.0931x; 1.0513x over previous
import functools

import jax
import jax.numpy as jnp
from jax import lax
from jax.experimental import pallas as pl
from jax.experimental.pallas import tpu as pltpu
from jax.experimental.pallas import tpu_sc as plsc

K = 512
C = 32
N = 4096
BLK = 1024
GRID = N // BLK

NC = 2
NS = 16
NW = NC * NS
BPW = N // NW

_BF = jnp.bfloat16
_F = jnp.float32


def _bdot(a, b):
    return lax.dot_general(a, b, (((1,), (0,)), ((), ())),
                           preferred_element_type=_F)


def _argmin_body(x_ref, e_ref, idx_ref,
                 ecat_s, eh_s, em_s, el_s, e2_s):
    @pl.when(pl.program_id(0) == 0)
    def _prep():
        e = e_ref[...]
        et = e.T
        eth = et.astype(_BF)
        etl = (et - eth.astype(_F)).astype(_BF)
        ecat_s[...] = jnp.concatenate([eth, etl, eth], axis=0)
        eh = e.astype(_BF)
        r = e - eh.astype(_F)
        em = r.astype(_BF)
        el = (r - em.astype(_F)).astype(_BF)
        eh_s[...] = eh
        em_s[...] = em
        el_s[...] = el
        e2_s[...] = jnp.sum(et * et, axis=0)[None, :]

    x = x_ref[...]
    xh = x.astype(_BF)
    xl = (x - xh.astype(_F)).astype(_BF)
    xcat = jnp.concatenate([xh, xh, xl], axis=1)
    s = _bdot(xcat, ecat_s[...])
    p = e2_s[...] - 2.0 * s
    kio = lax.broadcasted_iota(jnp.int32, (BLK, K), 1)
    v1 = jnp.min(p, axis=1)
    i1 = jnp.min(jnp.where(p == v1[:, None], kio, K), axis=1)
    m1 = kio == i1[:, None]
    p2m = jnp.where(m1, jnp.inf, p)
    v2 = jnp.min(p2m, axis=1)
    i2 = jnp.min(jnp.where(p2m == v2[:, None], kio, K), axis=1)
    m2 = kio == i2[:, None]
    mb = jnp.concatenate([m1.astype(_BF), m2.astype(_BF)], axis=0)
    eh = eh_s[...]
    em = em_s[...]
    el = el_s[...]
    e12 = (_bdot(mb, eh) + _bdot(mb, em)) + _bdot(mb, el)
    e1 = e12[:BLK]
    e2v = e12[BLK:]
    d1 = jnp.sum((x - e1) ** 2, axis=1)
    d2 = jnp.sum((x - e2v) ** 2, axis=1)
    pick = jnp.where(d2 < d1, i2, jnp.where(d1 < d2, i1, jnp.minimum(i1, i2)))
    idx_ref[0, 0, :] = pick


def _nearest_indices(flat, embedding, interpret=False):
    return pl.pallas_call(
        _argmin_body,
        grid=(GRID,),
        in_specs=[
            pl.BlockSpec((BLK, C), lambda i: (i, 0)),
            pl.BlockSpec((K, C), lambda i: (0, 0)),
        ],
        out_specs=pl.BlockSpec((1, 1, BLK), lambda i: (i, 0, 0)),
        out_shape=jax.ShapeDtypeStruct((GRID, 1, BLK), jnp.int32),
        scratch_shapes=[
            pltpu.VMEM((3 * C, K), _BF),
            pltpu.VMEM((K, C), _BF),
            pltpu.VMEM((K, C), _BF),
            pltpu.VMEM((K, C), _BF),
            pltpu.VMEM((1, K), _F),
        ],
        interpret=interpret,
    )(flat, embedding)


def _sc_gather_body(emb_hbm, idx_hbm, out_hbm, idx_v, rows_v, sem):
    wid = lax.axis_index("s") * NC + lax.axis_index("c")
    base = wid * BPW
    pltpu.sync_copy(idx_hbm.at[pl.ds(base, BPW)], idx_v)
    pltpu.async_copy(emb_hbm.at[idx_v], rows_v, sem).wait()
    pltpu.sync_copy(rows_v, out_hbm.at[pl.ds(base, BPW)])


@functools.cache
def _sc_gather():
    return pl.kernel(
        _sc_gather_body,
        out_type=jax.ShapeDtypeStruct((N, C), jnp.float32),
        mesh=plsc.VectorSubcoreMesh(core_axis_name="c", subcore_axis_name="s",
                                    num_cores=NC, num_subcores=NS),
        scratch_types=[
            pltpu.VMEM((BPW,), jnp.int32),
            pltpu.VMEM((BPW, C), jnp.float32),
            pltpu.SemaphoreType.DMA,
        ],
        compiler_params=pltpu.CompilerParams(use_tc_tiling_on_sc=False),
    )


def kernel(input, embedding):
    B, w, h, c = input.shape
    flat = input.reshape(B * w * h, c)
    idx = _nearest_indices(flat, embedding).reshape(-1)
    q = _sc_gather()(embedding, idx)
    return q.reshape(B, w, h, c)


# speedup vs baseline: 1.2086x; 1.1057x over previous
import functools

import jax
import jax.numpy as jnp
from jax import lax
from jax.experimental import pallas as pl
from jax.experimental.pallas import tpu as pltpu
from jax.experimental.pallas import tpu_sc as plsc

K = 512
C = 32
N = 4096
BLK = 1024
GRID = N // BLK

NC = 2
NS = 16
NW = NC * NS
BPW = N // NW

_BF = jnp.bfloat16
_F = jnp.float32


def _bdot(a, b):
    return lax.dot_general(a, b, (((1,), (0,)), ((), ())),
                           preferred_element_type=_F)


def _argmin_body(x_ref, e_ref, idx_ref,
                 ecat_s, e3_s, e2_s):
    @pl.when(pl.program_id(0) == 0)
    def _prep():
        e = e_ref[...]
        et = e.T
        eth = et.astype(_BF)
        etl = (et - eth.astype(_F)).astype(_BF)
        ecat_s[...] = jnp.concatenate([eth, etl, eth], axis=0)
        eh = e.astype(_BF)
        r = e - eh.astype(_F)
        em = r.astype(_BF)
        el = (r - em.astype(_F)).astype(_BF)
        e3_s[...] = jnp.concatenate([eh, em, el], axis=1)
        e2_s[...] = jnp.sum(et * et, axis=0)[None, :]

    x = x_ref[...]
    xh = x.astype(_BF)
    xl = (x - xh.astype(_F)).astype(_BF)
    xcat = jnp.concatenate([xh, xh, xl], axis=1)
    s = _bdot(xcat, ecat_s[...])
    p = e2_s[...] - 2.0 * s
    kio = lax.broadcasted_iota(jnp.int32, (BLK, K), 1)
    v1 = jnp.min(p, axis=1)
    i1 = jnp.min(jnp.where(p == v1[:, None], kio, K), axis=1)
    m1 = kio == i1[:, None]
    p2m = jnp.where(m1, jnp.inf, p)
    v2 = jnp.min(p2m, axis=1)
    i2 = jnp.min(jnp.where(p2m == v2[:, None], kio, K), axis=1)
    m2 = kio == i2[:, None]
    mb = jnp.concatenate([m1.astype(_BF), m2.astype(_BF)], axis=0)
    g = _bdot(mb, e3_s[...])
    e12 = (g[:, :C] + g[:, C:2 * C]) + g[:, 2 * C:]
    e1 = e12[:BLK]
    e2v = e12[BLK:]
    d1 = jnp.sum((x - e1) ** 2, axis=1)
    d2 = jnp.sum((x - e2v) ** 2, axis=1)
    pick = jnp.where(d2 < d1, i2, jnp.where(d1 < d2, i1, jnp.minimum(i1, i2)))
    idx_ref[0, 0, :] = pick


def _nearest_indices(flat, embedding, interpret=False):
    return pl.pallas_call(
        _argmin_body,
        grid=(GRID,),
        in_specs=[
            pl.BlockSpec((BLK, C), lambda i: (i, 0)),
            pl.BlockSpec((K, C), lambda i: (0, 0)),
        ],
        out_specs=pl.BlockSpec((1, 1, BLK), lambda i: (i, 0, 0)),
        out_shape=jax.ShapeDtypeStruct((GRID, 1, BLK), jnp.int32),
        scratch_shapes=[
            pltpu.VMEM((3 * C, K), _BF),
            pltpu.VMEM((K, 3 * C), _BF),
            pltpu.VMEM((1, K), _F),
        ],
        interpret=interpret,
    )(flat, embedding)


def _sc_gather_body(emb_hbm, idx_hbm, out_hbm, idx_v, rows_v, sem):
    wid = lax.axis_index("s") * NC + lax.axis_index("c")
    base = wid * BPW
    pltpu.sync_copy(idx_hbm.at[pl.ds(base, BPW)], idx_v)
    pltpu.async_copy(emb_hbm.at[idx_v], rows_v, sem).wait()
    pltpu.sync_copy(rows_v, out_hbm.at[pl.ds(base, BPW)])


@functools.cache
def _sc_gather():
    return pl.kernel(
        _sc_gather_body,
        out_type=jax.ShapeDtypeStruct((N, C), jnp.float32),
        mesh=plsc.VectorSubcoreMesh(core_axis_name="c", subcore_axis_name="s",
                                    num_cores=NC, num_subcores=NS),
        scratch_types=[
            pltpu.VMEM((BPW,), jnp.int32),
            pltpu.VMEM((BPW, C), jnp.float32),
            pltpu.SemaphoreType.DMA,
        ],
        compiler_params=pltpu.CompilerParams(use_tc_tiling_on_sc=False),
    )


def kernel(input, embedding):
    B, w, h, c = input.shape
    flat = input.reshape(B * w * h, c)
    idx = _nearest_indices(flat, embedding).reshape(-1)
    q = _sc_gather()(embedding, idx)
    return q.reshape(B, w, h, c)
